# Initial kernel scaffold; baseline (speedup 1.0000x reference)
import jax
import jax.numpy as jnp
from jax.experimental import pallas as pl


def kernel(x, edge_index, edge_vals, weight):
    raise NotImplementedError("write your pallas kernel here")


# speedup vs baseline: 4.5290x; 4.5290x over previous
import functools

import jax
import jax.numpy as jnp
from jax import lax
from jax.experimental import pallas as pl
from jax.experimental.pallas import tpu as pltpu
from jax.experimental.pallas import tpu_sc as plsc

NC = 2
NS = 16
L = 16


def _spmm_sc(x, row, col, vals, n, d, e):
    nw = NC * NS
    epw = e // nw
    be = 80
    nblk = epw // be
    n_pad = -(-n // (NS * 8)) * (NS * 8)
    rpt = n_pad // NS
    assert epw * nw == e and nblk * be == epw and d % L == 0
    zr = 8

    mesh = plsc.VectorSubcoreMesh(core_axis_name="c", subcore_axis_name="s")

    @functools.partial(
        pl.kernel,
        out_type=jax.ShapeDtypeStruct((NC, n_pad, d), jnp.float32),
        mesh=mesh,
        scratch_types=[
            pltpu.VMEM((be,), jnp.int32),
            pltpu.VMEM((be,), jnp.int32),
            pltpu.VMEM((be,), jnp.float32),
            pltpu.VMEM((be, d), jnp.float32),
            pltpu.VMEM((zr, d), jnp.float32),
            pltpu.VMEM_SHARED((n_pad, d), jnp.float32),
            pltpu.SemaphoreType.DMA,
        ],
    )
    def spmm(x_hbm, row_hbm, col_hbm, vals_hbm, out_hbm,
             col_v, row_v, vals_v, rows_v, zbuf, acc, sem):
        c = lax.axis_index("c")
        s = lax.axis_index("s")

        def zero_row(r, carry):
            for j in range(d // L):
                zbuf[r, pl.ds(j * L, L)] = jnp.zeros((L,), jnp.float32)
            return carry
        lax.fori_loop(0, zr, zero_row, 0)

        def zero_chunk(k, carry):
            pltpu.sync_copy(zbuf, acc.at[pl.ds(s * rpt + k * zr, zr)])
            return carry
        lax.fori_loop(0, rpt // zr, zero_chunk, 0)
        plsc.subcore_barrier()

        base_e = (c * NS + s) * epw

        def block(b, carry):
            base = base_e + b * be
            pltpu.sync_copy(col_hbm.at[pl.ds(base, be)], col_v)
            pltpu.sync_copy(row_hbm.at[pl.ds(base, be)], row_v)
            pltpu.sync_copy(vals_hbm.at[pl.ds(base, be)], vals_v)
            pltpu.async_copy(x_hbm.at[col_v], rows_v, sem).wait()

            def scale(g, carry2):
                vv = vals_v[pl.ds(g * L, L)]
                for t in range(L):
                    splat = vv[t]
                    i = g * L + t
                    for j in range(d // L):
                        sl = pl.ds(j * L, L)
                        rows_v[i, sl] = rows_v[i, sl] * splat
                return carry2
            lax.fori_loop(0, be // L, scale, 0)

            pltpu.sync_copy(rows_v, acc.at[row_v], add=True)
            return carry
        lax.fori_loop(0, nblk, block, 0)
        plsc.subcore_barrier()

        pltpu.sync_copy(acc.at[pl.ds(s * rpt, rpt)],
                        out_hbm.at[c, pl.ds(s * rpt, rpt)])

    return spmm(x, row, col, vals)


def _matmul_tc(z2, weight, n, d_in, d_out):
    bn = 2000
    assert n % bn == 0

    def body(z_ref, w_ref, o_ref):
        z = z_ref[0] + z_ref[1]
        o_ref[...] = jnp.dot(z, w_ref[...], preferred_element_type=jnp.float32)

    return pl.pallas_call(
        body,
        grid=(n // bn,),
        in_specs=[
            pl.BlockSpec((2, bn, d_in), lambda i: (0, i, 0)),
            pl.BlockSpec((d_in, d_out), lambda i: (0, 0)),
        ],
        out_specs=pl.BlockSpec((bn, d_out), lambda i: (i, 0)),
        out_shape=jax.ShapeDtypeStruct((n, d_out), jnp.float32),
    )(z2, weight)


@jax.jit
def kernel(x, edge_index, edge_vals, weight):
    n, d_in = x.shape
    d_out = weight.shape[1]
    e = edge_index.shape[1]
    row = edge_index[0].astype(jnp.int32)
    col = edge_index[1].astype(jnp.int32)
    z2 = _spmm_sc(x, row, col, edge_vals, n, d_in, e)
    return _matmul_tc(z2, weight, n, d_in, d_out)
